# baseline (speedup 1.0000x reference)
import dataclasses

import jax
import jax.numpy as jnp
from jax.experimental import pallas as pl
from jax.experimental.pallas import tpu as pltpu
from jax.experimental.pallas import tpu_sc as plsc

N_BLK = 1024
K_PAD = 8
LANES = 128
SUBS = 8

SC_SUBCORES = 32
SC_LANES = 16


def _chamfer_body(p1_ref, p2t_ref, p1b_ref, p2tb_ref,
                  d1_ref, i1_ref, d2_ref, i2_ref):
    n = pl.program_id(1)
    n_blk = d1_ref.shape[1]
    m = p2t_ref.shape[2]
    big = jnp.float32(4 * 1024 * 1024)

    p1 = p1_ref[0]
    p2t = p2t_ref[0]
    a_x = p1[:, 0:1]
    a_y = p1[:, 1:2]
    a_z = p1[:, 2:3]
    b_x = p2t[0:1, :]
    b_y = p2t[1:2, :]
    b_z = p2t[2:3, :]

    neg2cross = jax.lax.dot_general(
        p1b_ref[0], p2tb_ref[0],
        dimension_numbers=(((1,), (0,)), ((), ())),
        preferred_element_type=jnp.float32,
    )
    x1sq = a_x * a_x + a_y * a_y + a_z * a_z
    x2sq = b_x * b_x + b_y * b_y + b_z * b_z
    d = (x1sq + x2sq) + neg2cross

    curv = d[:, 0:LANES]
    curj = jnp.zeros((n_blk, LANES), jnp.float32)
    for j in range(1, m // LANES):
        blk = d[:, j * LANES:(j + 1) * LANES]
        mask = blk < curv
        curv = jnp.where(mask, blk, curv)
        curj = jnp.where(mask, jnp.float32(j), curj)
    dmin = jnp.min(curv, axis=1, keepdims=True)
    lane = jax.lax.broadcasted_iota(
        jnp.int32, (n_blk, LANES), 1).astype(jnp.float32)
    mfull = curj * jnp.float32(LANES) + lane
    cand = jnp.where(curv == dmin, mfull, big)
    imin = jnp.min(cand, axis=1, keepdims=True)
    d1_ref[0] = dmin
    i1_ref[0] = imin.astype(jnp.int32)

    cv = d[0:SUBS, :]
    cr = jnp.zeros((SUBS, m), jnp.float32)
    for r in range(1, n_blk // SUBS):
        blk = d[r * SUBS:(r + 1) * SUBS, :]
        mask = blk < cv
        cv = jnp.where(mask, blk, cv)
        cr = jnp.where(mask, jnp.float32(r), cr)
    sub = jax.lax.broadcasted_iota(
        jnp.int32, (SUBS, m), 0).astype(jnp.float32)
    nfull = cr * jnp.float32(SUBS) + sub
    for shift in (4, 2, 1):
        rv = pltpu.roll(cv, shift, 0)
        rn = pltpu.roll(nfull, shift, 0)
        lt = rv < cv
        eq = rv == cv
        nfull = jnp.where(lt, rn, jnp.where(eq, jnp.minimum(nfull, rn), nfull))
        cv = jnp.where(lt, rv, cv)
    cmin = cv[0:1, :]
    cidx = nfull[0:1, :] + jnp.float32(n_blk) * n.astype(jnp.float32)

    @pl.when(n == 0)
    def _():
        d2_ref[0] = cmin
        i2_ref[0] = cidx.astype(jnp.int32)

    @pl.when(n != 0)
    def _():
        prev_d = d2_ref[0]
        take_new = cmin < prev_d
        d2_ref[0] = jnp.where(take_new, cmin, prev_d)
        i2_ref[0] = jnp.where(take_new, cidx.astype(jnp.int32), i2_ref[0])


def _tc_chamfer(points1, points2, p2t):
    B, N, D = points1.shape
    M = points2.shape[1]
    pad = [(0, 0), (0, 0), (0, K_PAD - D)]
    p1b = jnp.pad((-2.0 * points1).astype(jnp.bfloat16), pad)
    p2tb = jnp.pad(p2t.astype(jnp.bfloat16),
                   [(0, 0), (0, K_PAD - D), (0, 0)])

    return pl.pallas_call(
        _chamfer_body,
        grid=(B, N // N_BLK),
        in_specs=[
            pl.BlockSpec((1, N_BLK, D), lambda b, n: (b, n, 0)),
            pl.BlockSpec((1, D, M), lambda b, n: (b, 0, 0)),
            pl.BlockSpec((1, N_BLK, K_PAD), lambda b, n: (b, n, 0)),
            pl.BlockSpec((1, K_PAD, M), lambda b, n: (b, 0, 0)),
        ],
        out_specs=[
            pl.BlockSpec((1, N_BLK, 1), lambda b, n: (b, n, 0)),
            pl.BlockSpec((1, N_BLK, 1), lambda b, n: (b, n, 0)),
            pl.BlockSpec((1, 1, M), lambda b, n: (b, 0, 0)),
            pl.BlockSpec((1, 1, M), lambda b, n: (b, 0, 0)),
        ],
        out_shape=[
            jax.ShapeDtypeStruct((B, N, 1), jnp.float32),
            jax.ShapeDtypeStruct((B, N, 1), jnp.int32),
            jax.ShapeDtypeStruct((B, 1, M), jnp.float32),
            jax.ShapeDtypeStruct((B, 1, M), jnp.int32),
        ],
    )(points1, p2t, p1b, p2tb)


def _rbf16_bits(v):
    i = jax.lax.bitcast_convert_type(v, jnp.int32)
    r = (i + jnp.int32(0x7FFF) + ((i >> 16) & 1)) & jnp.int32(-65536)
    return jax.lax.bitcast_convert_type(r, jnp.float32)


def _sc_chamfer(rp1, x1sq, rp2, x2sq):
    M = rp1.shape[1]
    rows_per_sub = M // SC_SUBCORES
    n_chunks = M // SC_LANES
    groups = rows_per_sub // SC_LANES
    big = jnp.float32(4 * 1024 * 1024)

    mesh = plsc.VectorSubcoreMesh(core_axis_name="c", subcore_axis_name="s")
    cp = pltpu.CompilerParams()
    if "needs_layout_passes" in pltpu.CompilerParams.__dataclass_fields__:
        cp = dataclasses.replace(cp, needs_layout_passes=False)

    @pl.kernel(
        compiler_params=cp,
        out_type=[
            jax.ShapeDtypeStruct((M,), jnp.float32),
            jax.ShapeDtypeStruct((M,), jnp.float32),
            jax.ShapeDtypeStruct((SC_SUBCORES, M), jnp.float32),
            jax.ShapeDtypeStruct((SC_SUBCORES, M), jnp.float32),
        ],
        mesh=mesh,
        scratch_types=[
            pltpu.VMEM((3, M), jnp.float32),
            pltpu.VMEM((M,), jnp.float32),
            pltpu.VMEM((3, M), jnp.float32),
            pltpu.VMEM((M,), jnp.float32),
            pltpu.VMEM((M,), jnp.float32),
            pltpu.VMEM((M,), jnp.float32),
            pltpu.VMEM((rows_per_sub,), jnp.float32),
            pltpu.VMEM((rows_per_sub,), jnp.float32),
        ],
    )
    def sc_kernel(rp1_hbm, x1_hbm, rp2_hbm, x2_hbm,
                  d1_hbm, i1_hbm, cm_hbm, ci_hbm,
                  rp1v, x1v, rp2v, x2v, cmv, civ, d1b, i1b):
        sg = jax.lax.axis_index("c") * 16 + jax.lax.axis_index("s")

        pltpu.sync_copy(rp1_hbm, rp1v)
        pltpu.sync_copy(x1_hbm, x1v)
        pltpu.sync_copy(rp2_hbm, rp2v)
        pltpu.sync_copy(x2_hbm, x2v)

        lane_i = jax.lax.broadcasted_iota(jnp.int32, (SC_LANES,), 0)
        lane_f = lane_i.astype(jnp.float32)
        zeros_i = jnp.zeros((SC_LANES,), jnp.int32)
        ones_i = jnp.ones((SC_LANES,), jnp.int32)
        big_v = jnp.full((SC_LANES,), big, jnp.float32)

        @pl.loop(0, n_chunks)
        def _(c):
            sl = pl.ds(c * SC_LANES, SC_LANES)
            cmv[sl] = big_v
            civ[sl] = jnp.zeros((SC_LANES,), jnp.float32)

        @pl.loop(0, groups)
        def _(g):
            def row_step(k, gcarry):
                dminv, iminv = gcarry
                row = (sg * rows_per_sub + g * SC_LANES) + k
                rsplat = jnp.full((SC_LANES,), row, jnp.int32)
                axs = plsc.load_gather(rp1v, [zeros_i, rsplat])
                ays = plsc.load_gather(rp1v, [ones_i, rsplat])
                azs = plsc.load_gather(rp1v, [ones_i + ones_i, rsplat])
                x1s = plsc.load_gather(x1v, [rsplat])
                nfs = rsplat.astype(jnp.float32)

                def chunk(c, carry):
                    curv, curj = carry
                    sl = pl.ds(c * SC_LANES, SC_LANES)
                    t = (axs * rp2v[0, sl] + ays * rp2v[1, sl]) \
                        + azs * rp2v[2, sl]
                    dd = (x1s + x2v[sl]) + t
                    m1 = dd < curv
                    curv = jnp.where(m1, dd, curv)
                    jf = jnp.full((SC_LANES,), c.astype(jnp.float32),
                                  jnp.float32)
                    curj = jnp.where(m1, jf, curj)
                    cm = cmv[sl]
                    m2 = dd < cm
                    cmv[sl] = jnp.where(m2, dd, cm)
                    civ[sl] = jnp.where(m2, nfs, civ[sl])
                    return curv, curj

                init = (big_v, jnp.zeros((SC_LANES,), jnp.float32))
                curv, curj = jax.lax.fori_loop(0, n_chunks, chunk, init)

                dmin = jnp.min(curv)
                dms = jnp.full((SC_LANES,), dmin, jnp.float32)
                mf = curj * jnp.float32(SC_LANES) + lane_f
                cand = jnp.where(curv == dms, mf, big_v)
                imin = jnp.min(cand)
                here = lane_i == jnp.full((SC_LANES,), k, jnp.int32)
                dminv = jnp.where(here, dms, dminv)
                iminv = jnp.where(
                    here, jnp.full((SC_LANES,), imin, jnp.float32), iminv)
                return dminv, iminv

            ginit = (big_v, big_v)
            dminv, iminv = jax.lax.fori_loop(0, SC_LANES, row_step, ginit)
            gsl = pl.ds(g * SC_LANES, SC_LANES)
            d1b[gsl] = dminv
            i1b[gsl] = iminv

        out_sl = pl.ds(sg * rows_per_sub, rows_per_sub)
        pltpu.sync_copy(d1b, d1_hbm.at[out_sl])
        pltpu.sync_copy(i1b, i1_hbm.at[out_sl])
        pltpu.sync_copy(cmv, cm_hbm.at[sg])
        pltpu.sync_copy(civ, ci_hbm.at[sg])

    return sc_kernel(rp1, x1sq, rp2, x2sq)


def _merge_body(cm_ref, ci_ref, d2_ref, i2_ref):
    cv = cm_ref[0, 0:SUBS, :]
    ni = ci_ref[0, 0:SUBS, :]
    for r in range(1, SC_SUBCORES // SUBS):
        bv = cm_ref[0, r * SUBS:(r + 1) * SUBS, :]
        bi = ci_ref[0, r * SUBS:(r + 1) * SUBS, :]
        mask = bv < cv
        cv = jnp.where(mask, bv, cv)
        ni = jnp.where(mask, bi, ni)
    for shift in (4, 2, 1):
        rv = pltpu.roll(cv, shift, 0)
        rn = pltpu.roll(ni, shift, 0)
        lt = rv < cv
        eq = rv == cv
        ni = jnp.where(lt, rn, jnp.where(eq, jnp.minimum(ni, rn), ni))
        cv = jnp.where(lt, rv, cv)
    d2_ref[0] = cv[0:1, :]
    i2_ref[0] = ni[0:1, :].astype(jnp.int32)


def _merge_cols(cm, ci):
    M = cm.shape[1]
    d2, i2 = pl.pallas_call(
        _merge_body,
        in_specs=[
            pl.BlockSpec((1, SC_SUBCORES, M), lambda: (0, 0, 0)),
            pl.BlockSpec((1, SC_SUBCORES, M), lambda: (0, 0, 0)),
        ],
        out_specs=[
            pl.BlockSpec((1, 1, M), lambda: (0, 0, 0)),
            pl.BlockSpec((1, 1, M), lambda: (0, 0, 0)),
        ],
        out_shape=[
            jax.ShapeDtypeStruct((1, 1, M), jnp.float32),
            jax.ShapeDtypeStruct((1, 1, M), jnp.int32),
        ],
    )(cm.reshape(1, SC_SUBCORES, M), ci.reshape(1, SC_SUBCORES, M))
    return d2[0, 0], i2[0, 0]


def kernel(points1, points2):
    B, N, D = points1.shape
    M = points2.shape[1]
    p2t = points2.transpose(0, 2, 1)
    p1t = points1.transpose(0, 2, 1)

    b_tc = B - 1
    d1_tc, i1_tc, d2_tc, i2_tc = _tc_chamfer(
        points1[:b_tc], points2[:b_tc], p2t[:b_tc])

    p1l = p1t[b_tc]
    p2l = p2t[b_tc]
    rp1 = _rbf16_bits(-2.0 * p1l)
    rp2 = _rbf16_bits(p2l)
    x1sq = (p1l[0] * p1l[0] + p1l[1] * p1l[1]) + p1l[2] * p1l[2]
    x2sq = (p2l[0] * p2l[0] + p2l[1] * p2l[1]) + p2l[2] * p2l[2]
    d1s, i1sf, cmall, ciall = _sc_chamfer(rp1, x1sq, rp2, x2sq)
    d2s, i2s = _merge_cols(cmall, ciall)

    idx1 = jnp.concatenate(
        [i1_tc[..., 0], i1sf.astype(jnp.int32)[None]], axis=0)
    idx2 = jnp.concatenate([i2_tc[:, 0, :], i2s[None]], axis=0)
    dist1 = jnp.concatenate([d1_tc[..., 0], d1s[None]], axis=0)
    dist2 = jnp.concatenate([d2_tc[:, 0, :], d2s[None]], axis=0)
    return (idx1, idx2, dist1, dist2)


# speedup vs baseline: 1.6499x; 1.6499x over previous
import jax
import jax.numpy as jnp
from jax.experimental import pallas as pl
from jax.experimental.pallas import tpu as pltpu

N_BLK = 2048
K_PAD = 8
LANES = 128
SUBS = 8


def _chamfer_body(p1_ref, p2t_ref, p1b_ref, p2tb_ref,
                  d1_ref, i1_ref, d2_ref, i2_ref):
    n = pl.program_id(1)
    n_blk = d1_ref.shape[1]
    m = p2t_ref.shape[2]
    big = jnp.float32(4 * 1024 * 1024)

    p1 = p1_ref[0]
    p2t = p2t_ref[0]
    a_x = p1[:, 0:1]
    a_y = p1[:, 1:2]
    a_z = p1[:, 2:3]
    b_x = p2t[0:1, :]
    b_y = p2t[1:2, :]
    b_z = p2t[2:3, :]

    neg2cross = jax.lax.dot_general(
        p1b_ref[0], p2tb_ref[0],
        dimension_numbers=(((1,), (0,)), ((), ())),
        preferred_element_type=jnp.float32,
    )
    x1sq = a_x * a_x + a_y * a_y + a_z * a_z
    x2sq = b_x * b_x + b_y * b_y + b_z * b_z
    d = (x1sq + x2sq) + neg2cross

    curv = d[:, 0:LANES]
    curj = jnp.zeros((n_blk, LANES), jnp.float32)
    for j in range(1, m // LANES):
        blk = d[:, j * LANES:(j + 1) * LANES]
        mask = blk < curv
        curv = jnp.where(mask, blk, curv)
        curj = jnp.where(mask, jnp.float32(j), curj)
    dmin = jnp.min(curv, axis=1, keepdims=True)
    lane = jax.lax.broadcasted_iota(jnp.int32, (n_blk, LANES), 1).astype(jnp.float32)
    mfull = curj * jnp.float32(LANES) + lane
    cand = jnp.where(curv == dmin, mfull, big)
    imin = jnp.min(cand, axis=1, keepdims=True)
    d1_ref[0] = dmin
    i1_ref[0] = imin.astype(jnp.int32)

    cv = d[0:SUBS, :]
    cr = jnp.zeros((SUBS, m), jnp.float32)
    for r in range(1, n_blk // SUBS):
        blk = d[r * SUBS:(r + 1) * SUBS, :]
        mask = blk < cv
        cv = jnp.where(mask, blk, cv)
        cr = jnp.where(mask, jnp.float32(r), cr)
    sub = jax.lax.broadcasted_iota(jnp.int32, (SUBS, m), 0).astype(jnp.float32)
    nfull = cr * jnp.float32(SUBS) + sub
    for shift in (4, 2, 1):
        rv = pltpu.roll(cv, shift, 0)
        rn = pltpu.roll(nfull, shift, 0)
        lt = rv < cv
        eq = rv == cv
        nfull = jnp.where(lt, rn, jnp.where(eq, jnp.minimum(nfull, rn), nfull))
        cv = jnp.where(lt, rv, cv)
    cmin = cv[0:1, :]
    cidx = nfull[0:1, :] + jnp.float32(n_blk) * n.astype(jnp.float32)

    @pl.when(n == 0)
    def _():
        d2_ref[0] = cmin
        i2_ref[0] = cidx.astype(jnp.int32)

    @pl.when(n != 0)
    def _():
        prev_d = d2_ref[0]
        take_new = cmin < prev_d
        d2_ref[0] = jnp.where(take_new, cmin, prev_d)
        i2_ref[0] = jnp.where(take_new, cidx.astype(jnp.int32), i2_ref[0])


def kernel(points1, points2):
    B, N, D = points1.shape
    M = points2.shape[1]
    p2t = points2.transpose(0, 2, 1)

    pad = [(0, 0), (0, 0), (0, K_PAD - D)]
    p1b = jnp.pad((-2.0 * points1).astype(jnp.bfloat16), pad)
    p2tb = jnp.pad(p2t.astype(jnp.bfloat16),
                   [(0, 0), (0, K_PAD - D), (0, 0)])

    d1, i1, d2, i2 = pl.pallas_call(
        _chamfer_body,
        grid=(B, N // N_BLK),
        in_specs=[
            pl.BlockSpec((1, N_BLK, D), lambda b, n: (b, n, 0)),
            pl.BlockSpec((1, D, M), lambda b, n: (b, 0, 0)),
            pl.BlockSpec((1, N_BLK, K_PAD), lambda b, n: (b, n, 0)),
            pl.BlockSpec((1, K_PAD, M), lambda b, n: (b, 0, 0)),
        ],
        out_specs=[
            pl.BlockSpec((1, N_BLK, 1), lambda b, n: (b, n, 0)),
            pl.BlockSpec((1, N_BLK, 1), lambda b, n: (b, n, 0)),
            pl.BlockSpec((1, 1, M), lambda b, n: (b, 0, 0)),
            pl.BlockSpec((1, 1, M), lambda b, n: (b, 0, 0)),
        ],
        out_shape=[
            jax.ShapeDtypeStruct((B, N, 1), jnp.float32),
            jax.ShapeDtypeStruct((B, N, 1), jnp.int32),
            jax.ShapeDtypeStruct((B, 1, M), jnp.float32),
            jax.ShapeDtypeStruct((B, 1, M), jnp.int32),
        ],
    )(points1, p2t, p1b, p2tb)

    return (i1[..., 0], i2[:, 0, :], d1[..., 0], d2[:, 0, :])


# speedup vs baseline: 1.6525x; 1.0016x over previous
import jax
import jax.numpy as jnp
from jax.experimental import pallas as pl
from jax.experimental.pallas import tpu as pltpu

N_BLK = 2048
K_PAD = 8
LANES = 128
SUBS = 8


def _chamfer_body(p1_ref, p2t_ref, p1b_ref, p2tb_ref,
                  d1_ref, i1_ref, d2_ref, i2_ref):
    n = pl.program_id(1)
    n_blk = d1_ref.shape[1]
    m = p2t_ref.shape[2]
    big = jnp.float32(4 * 1024 * 1024)

    p1 = p1_ref[0]
    p2t = p2t_ref[0]
    a_x = p1[:, 0:1]
    a_y = p1[:, 1:2]
    a_z = p1[:, 2:3]
    b_x = p2t[0:1, :]
    b_y = p2t[1:2, :]
    b_z = p2t[2:3, :]

    neg2cross = jax.lax.dot_general(
        p1b_ref[0], p2tb_ref[0],
        dimension_numbers=(((1,), (0,)), ((), ())),
        preferred_element_type=jnp.float32,
    )
    x1sq = a_x * a_x + a_y * a_y + a_z * a_z
    x2sq = b_x * b_x + b_y * b_y + b_z * b_z
    d = (x1sq + x2sq) + neg2cross

    curv = d[:, 0:LANES]
    curj = jnp.zeros((n_blk, LANES), jnp.float32)
    for j in range(1, m // LANES):
        blk = d[:, j * LANES:(j + 1) * LANES]
        mask = blk < curv
        curv = jnp.minimum(blk, curv)
        curj = jnp.where(mask, jnp.float32(j), curj)
    dmin = jnp.min(curv, axis=1, keepdims=True)
    lane = jax.lax.broadcasted_iota(jnp.int32, (n_blk, LANES), 1).astype(jnp.float32)
    mfull = curj * jnp.float32(LANES) + lane
    cand = jnp.where(curv == dmin, mfull, big)
    imin = jnp.min(cand, axis=1, keepdims=True)
    d1_ref[0] = dmin
    i1_ref[0] = imin.astype(jnp.int32)

    cv = d[0:SUBS, :]
    cr = jnp.zeros((SUBS, m), jnp.float32)
    for r in range(1, n_blk // SUBS):
        blk = d[r * SUBS:(r + 1) * SUBS, :]
        mask = blk < cv
        cv = jnp.minimum(blk, cv)
        cr = jnp.where(mask, jnp.float32(r), cr)
    sub = jax.lax.broadcasted_iota(jnp.int32, (SUBS, m), 0).astype(jnp.float32)
    nfull = cr * jnp.float32(SUBS) + sub
    for shift in (4, 2, 1):
        rv = pltpu.roll(cv, shift, 0)
        rn = pltpu.roll(nfull, shift, 0)
        lt = rv < cv
        eq = rv == cv
        nfull = jnp.where(lt, rn, jnp.where(eq, jnp.minimum(nfull, rn), nfull))
        cv = jnp.where(lt, rv, cv)
    cmin = cv[0:1, :]
    cidx = nfull[0:1, :] + jnp.float32(n_blk) * n.astype(jnp.float32)

    @pl.when(n == 0)
    def _():
        d2_ref[0] = cmin
        i2_ref[0] = cidx.astype(jnp.int32)

    @pl.when(n != 0)
    def _():
        prev_d = d2_ref[0]
        take_new = cmin < prev_d
        d2_ref[0] = jnp.where(take_new, cmin, prev_d)
        i2_ref[0] = jnp.where(take_new, cidx.astype(jnp.int32), i2_ref[0])


def kernel(points1, points2):
    B, N, D = points1.shape
    M = points2.shape[1]
    p2t = points2.transpose(0, 2, 1)

    pad = [(0, 0), (0, 0), (0, K_PAD - D)]
    p1b = jnp.pad((-2.0 * points1).astype(jnp.bfloat16), pad)
    p2tb = jnp.pad(p2t.astype(jnp.bfloat16),
                   [(0, 0), (0, K_PAD - D), (0, 0)])

    d1, i1, d2, i2 = pl.pallas_call(
        _chamfer_body,
        grid=(B, N // N_BLK),
        in_specs=[
            pl.BlockSpec((1, N_BLK, D), lambda b, n: (b, n, 0)),
            pl.BlockSpec((1, D, M), lambda b, n: (b, 0, 0)),
            pl.BlockSpec((1, N_BLK, K_PAD), lambda b, n: (b, n, 0)),
            pl.BlockSpec((1, K_PAD, M), lambda b, n: (b, 0, 0)),
        ],
        out_specs=[
            pl.BlockSpec((1, N_BLK, 1), lambda b, n: (b, n, 0)),
            pl.BlockSpec((1, N_BLK, 1), lambda b, n: (b, n, 0)),
            pl.BlockSpec((1, 1, M), lambda b, n: (b, 0, 0)),
            pl.BlockSpec((1, 1, M), lambda b, n: (b, 0, 0)),
        ],
        out_shape=[
            jax.ShapeDtypeStruct((B, N, 1), jnp.float32),
            jax.ShapeDtypeStruct((B, N, 1), jnp.int32),
            jax.ShapeDtypeStruct((B, 1, M), jnp.float32),
            jax.ShapeDtypeStruct((B, 1, M), jnp.int32),
        ],
    )(points1, p2t, p1b, p2tb)

    return (i1[..., 0], i2[:, 0, :], d1[..., 0], d2[:, 0, :])


# speedup vs baseline: 1.7856x; 1.0806x over previous
import jax
import jax.numpy as jnp
from jax.experimental import pallas as pl
from jax.experimental.pallas import tpu as pltpu

N_BLK = 2048
K_PAD = 8
LANES = 128
SUBS = 8


def _chamfer_body(x1s_ref, x2s_ref, p1b_ref, p2tb_ref,
                  d1_ref, i1_ref, d2_ref, i2_ref):
    n = pl.program_id(1)
    n_blk = d1_ref.shape[1]
    m = x2s_ref.shape[2]
    big = jnp.float32(4 * 1024 * 1024)

    neg2cross = jax.lax.dot_general(
        p1b_ref[0], p2tb_ref[0],
        dimension_numbers=(((1,), (0,)), ((), ())),
        preferred_element_type=jnp.float32,
    )
    d = (x1s_ref[0] + x2s_ref[0]) + neg2cross

    curv = d[:, 0:LANES]
    curj = jnp.zeros((n_blk, LANES), jnp.float32)
    for j in range(1, m // LANES):
        blk = d[:, j * LANES:(j + 1) * LANES]
        mask = blk < curv
        curv = jnp.minimum(blk, curv)
        curj = jnp.where(mask, jnp.float32(j), curj)
    dmin = jnp.min(curv, axis=1, keepdims=True)
    lane = jax.lax.broadcasted_iota(jnp.int32, (n_blk, LANES), 1).astype(jnp.float32)
    mfull = curj * jnp.float32(LANES) + lane
    cand = jnp.where(curv == dmin, mfull, big)
    imin = jnp.min(cand, axis=1, keepdims=True)
    d1_ref[0] = dmin
    i1_ref[0] = imin.astype(jnp.int32)

    cv = d[0:SUBS, :]
    cr = jnp.zeros((SUBS, m), jnp.float32)
    for r in range(1, n_blk // SUBS):
        blk = d[r * SUBS:(r + 1) * SUBS, :]
        mask = blk < cv
        cv = jnp.minimum(blk, cv)
        cr = jnp.where(mask, jnp.float32(r), cr)
    sub = jax.lax.broadcasted_iota(jnp.int32, (SUBS, m), 0).astype(jnp.float32)
    nfull = cr * jnp.float32(SUBS) + sub
    for shift in (4, 2, 1):
        rv = pltpu.roll(cv, shift, 0)
        rn = pltpu.roll(nfull, shift, 0)
        lt = rv < cv
        eq = rv == cv
        nfull = jnp.where(lt, rn, jnp.where(eq, jnp.minimum(nfull, rn), nfull))
        cv = jnp.where(lt, rv, cv)
    cmin = cv[0:1, :]
    cidx = nfull[0:1, :] + jnp.float32(n_blk) * n.astype(jnp.float32)

    @pl.when(n == 0)
    def _():
        d2_ref[0] = cmin
        i2_ref[0] = cidx.astype(jnp.int32)

    @pl.when(n != 0)
    def _():
        prev_d = d2_ref[0]
        take_new = cmin < prev_d
        d2_ref[0] = jnp.where(take_new, cmin, prev_d)
        i2_ref[0] = jnp.where(take_new, cidx.astype(jnp.int32), i2_ref[0])


def kernel(points1, points2):
    B, N, D = points1.shape
    M = points2.shape[1]
    p2t = points2.transpose(0, 2, 1)

    pad = [(0, 0), (0, 0), (0, K_PAD - D)]
    p1b = jnp.pad((-2.0 * points1).astype(jnp.bfloat16), pad)
    p2tb = jnp.pad(p2t.astype(jnp.bfloat16),
                   [(0, 0), (0, K_PAD - D), (0, 0)])
    x1s = jnp.sum(points1 * points1, axis=-1)[:, :, None]
    x2s = jnp.sum(points2 * points2, axis=-1)[:, None, :]

    d1, i1, d2, i2 = pl.pallas_call(
        _chamfer_body,
        grid=(B, N // N_BLK),
        in_specs=[
            pl.BlockSpec((1, N_BLK, 1), lambda b, n: (b, n, 0)),
            pl.BlockSpec((1, 1, M), lambda b, n: (b, 0, 0)),
            pl.BlockSpec((1, N_BLK, K_PAD), lambda b, n: (b, n, 0)),
            pl.BlockSpec((1, K_PAD, M), lambda b, n: (b, 0, 0)),
        ],
        out_specs=[
            pl.BlockSpec((1, N_BLK, 1), lambda b, n: (b, n, 0)),
            pl.BlockSpec((1, N_BLK, 1), lambda b, n: (b, n, 0)),
            pl.BlockSpec((1, 1, M), lambda b, n: (b, 0, 0)),
            pl.BlockSpec((1, 1, M), lambda b, n: (b, 0, 0)),
        ],
        out_shape=[
            jax.ShapeDtypeStruct((B, N, 1), jnp.float32),
            jax.ShapeDtypeStruct((B, N, 1), jnp.int32),
            jax.ShapeDtypeStruct((B, 1, M), jnp.float32),
            jax.ShapeDtypeStruct((B, 1, M), jnp.int32),
        ],
    )(x1s, x2s, p1b, p2tb)

    return (i1[..., 0], i2[:, 0, :], d1[..., 0], d2[:, 0, :])
